# Initial kernel scaffold; baseline (speedup 1.0000x reference)
import jax
import jax.numpy as jnp
from jax.experimental import pallas as pl


def kernel(h, h_up, h_down, x, x_up, x_down, b_up, b_down, cw, bu_w1, bu_b1, bu_w2, bu_b2, bd_w1, bd_b1, bd_w2, bd_b2, cm_w1, cm_b1, cm_w2, cm_b2, cu_w1, cu_b1, cu_w2, cu_b2, cd_w1, cd_b1, cd_w2, cd_b2):
    raise NotImplementedError("write your pallas kernel here")


# speedup vs baseline: 1.8684x; 1.8684x over previous
import functools

import jax
import jax.numpy as jnp
from jax import lax
from jax.experimental import pallas as pl
from jax.experimental.pallas import tpu as pltpu
from jax.experimental.pallas import tpu_sc as plsc

NC = 2
NS = 16
XP = 16
TW = 256


def _k1_body(h_ref, hs_ref, xpad_ref, xnbs_ref, wa_ref, wb_ref, b1_ref,
             w1c_ref, ta_ref, tb_ref):
    xa = xpad_ref[...]
    xb = xnbs_ref[0]
    x2a = jnp.sum(xa * xa, axis=-1, keepdims=True)
    x2b = jnp.sum(xb * xb, axis=-1, keepdims=True)
    pa = (jnp.dot(h_ref[...], wa_ref[0], preferred_element_type=jnp.float32)
          + b1_ref[0] + x2a * w1c_ref[0])
    qb = (jnp.dot(hs_ref[0], wb_ref[0], preferred_element_type=jnp.float32)
          + x2b * w1c_ref[0])
    z = jnp.zeros((pa.shape[0], TW - pa.shape[1] - xa.shape[1]), jnp.bfloat16)
    ta_ref[0] = jnp.concatenate(
        [pa.astype(jnp.bfloat16), xa.astype(jnp.bfloat16), z], axis=1)
    tb_ref[0] = jnp.concatenate(
        [qb.astype(jnp.bfloat16), xb.astype(jnp.bfloat16), z], axis=1)


def _make_tables(h, hs, xpad, xnbs, wa, wb, b1s, w1c, n, d):
    bn = 2000
    grid = (2, n // bn)
    return pl.pallas_call(
        _k1_body,
        grid=grid,
        in_specs=[
            pl.BlockSpec((bn, d), lambda c, i: (i, 0)),
            pl.BlockSpec((1, bn, d), lambda c, i: (c, i, 0)),
            pl.BlockSpec((bn, XP), lambda c, i: (i, 0)),
            pl.BlockSpec((1, bn, XP), lambda c, i: (c, i, 0)),
            pl.BlockSpec((1, d, d), lambda c, i: (c, 0, 0)),
            pl.BlockSpec((1, d, d), lambda c, i: (c, 0, 0)),
            pl.BlockSpec((1, 1, d), lambda c, i: (c, 0, 0)),
            pl.BlockSpec((1, 1, d), lambda c, i: (c, 0, 0)),
        ],
        out_specs=[
            pl.BlockSpec((1, bn, TW), lambda c, i: (c, i, 0)),
            pl.BlockSpec((1, bn, TW), lambda c, i: (c, i, 0)),
        ],
        out_shape=[
            jax.ShapeDtypeStruct((2, n, TW), jnp.bfloat16),
            jax.ShapeDtypeStruct((2, n, TW), jnp.bfloat16),
        ],
    )(h, hs, xpad, xnbs, wa, wb, b1s, w1c)


def _k2_body(e, k, ta, tb, ia, ib, ga, gb,
             idxa_v, idxb_v, bufa, bufb, sem):
    c = lax.axis_index("c")
    s = lax.axis_index("s")
    per_sub = e // NS
    chunks = per_sub // k

    def step(t, _):
        base = s * per_sub + t * k
        pltpu.sync_copy(ia.at[pl.ds(c * e + base, k)], idxa_v)
        pltpu.sync_copy(ib.at[pl.ds(c * e + base, k)], idxb_v)
        cp1 = pltpu.async_copy(ta.at[idxa_v], bufa, sem)
        cp2 = pltpu.async_copy(tb.at[idxb_v], bufb, sem)
        cp1.wait()
        cp2.wait()
        pltpu.sync_copy(bufa, ga.at[c, pl.ds(base, k)])
        pltpu.sync_copy(bufb, gb.at[c, pl.ds(base, k)])
        return _

    lax.fori_loop(0, chunks, step, None)


def _gather_stage(ta, tb, ia_off, ib_off, e):
    k = 400
    tw2 = TW // 2
    mesh = plsc.VectorSubcoreMesh(core_axis_name="c", subcore_axis_name="s")
    fn = pl.kernel(
        functools.partial(_k2_body, e, k),
        out_type=[
            jax.ShapeDtypeStruct((2, e, tw2), jnp.int32),
            jax.ShapeDtypeStruct((2, e, tw2), jnp.int32),
        ],
        mesh=mesh,
        scratch_types=[
            pltpu.VMEM((k,), jnp.int32),
            pltpu.VMEM((k,), jnp.int32),
            pltpu.VMEM((k, tw2), jnp.int32),
            pltpu.VMEM((k, tw2), jnp.int32),
            pltpu.SemaphoreType.DMA,
        ],
    )
    return fn(ta, tb, ia_off, ib_off)


def _k3_body(d, ga_ref, gb_ref, w1c_ref, w2_ref, b2_ref,
             cw1_ref, cb1_ref, cw2t_ref, cb2_ref, cws_ref, m_ref, w_ref):
    di = pl.program_id(0)
    a = ga_ref[0]
    b = gb_ref[0]
    ap = a[:, :d].astype(jnp.float32)
    bq = b[:, :d].astype(jnp.float32)
    xa = a[:, d:d + XP].astype(jnp.float32)
    xb = b[:, d:d + XP].astype(jnp.float32)
    sij = jnp.sum(xa * xb, axis=-1, keepdims=True)
    xd = xa - xb
    u = jnp.maximum(ap + bq - 2.0 * sij * w1c_ref[0], 0.0)
    m = jnp.dot(u.astype(jnp.bfloat16), w2_ref[0],
                preferred_element_type=jnp.float32) + b2_ref[0]
    g = jnp.maximum(
        jnp.dot(m.astype(jnp.bfloat16), cw1_ref[0],
                preferred_element_type=jnp.float32) + cb1_ref[0], 0.0)
    sp = jnp.sum(g * cw2t_ref[0], axis=-1, keepdims=True) + cb2_ref[di, 0]
    gate = jax.nn.sigmoid(sp)
    m_ref[0] = m
    wv = xd[:, :4] * (gate * cws_ref[di])
    w_ref[0] = wv.T


def _edge_mlp(ga, gb, w1c, w2b, b2s, cw1b, cb1s, cw2t, cb2s, cws, e, d):
    kb = 2560
    grid = (2, e // kb)
    return pl.pallas_call(
        functools.partial(_k3_body, d),
        grid=grid,
        in_specs=[
            pl.BlockSpec((1, kb, TW), lambda c, i: (c, i, 0)),
            pl.BlockSpec((1, kb, TW), lambda c, i: (c, i, 0)),
            pl.BlockSpec((1, 1, d), lambda c, i: (c, 0, 0)),
            pl.BlockSpec((1, d, d), lambda c, i: (c, 0, 0)),
            pl.BlockSpec((1, 1, d), lambda c, i: (c, 0, 0)),
            pl.BlockSpec((1, d, d), lambda c, i: (c, 0, 0)),
            pl.BlockSpec((1, 1, d), lambda c, i: (c, 0, 0)),
            pl.BlockSpec((1, 1, d), lambda c, i: (c, 0, 0)),
            pl.BlockSpec(memory_space=pltpu.SMEM),
            pl.BlockSpec(memory_space=pltpu.SMEM),
        ],
        out_specs=[
            pl.BlockSpec((1, kb, d), lambda c, i: (c, i, 0)),
            pl.BlockSpec((1, 4, kb), lambda c, i: (c, 0, i)),
        ],
        out_shape=[
            jax.ShapeDtypeStruct((2, e, d), jnp.float32),
            jax.ShapeDtypeStruct((2, 4, e), jnp.float32),
        ],
    )(ga, gb, w1c, w2b, b2s, cw1b, cb1s, cw2t, cb2s, cws)


def _k4m_body(half, rr, e, k, j, mm, ia, zm, outm,
              idx_v, idxt_v, m_v, accm):
    c = lax.axis_index("c")
    s = lax.axis_index("s")
    per_sub = e // NS
    chunks = per_sub // k
    zrows = rr // NS
    wrows = half // NS

    pltpu.sync_copy(zm.at[pl.ds(s * zrows, zrows)],
                    accm.at[pl.ds(s * zrows, zrows)])
    plsc.subcore_barrier()

    def step(t, _):
        base = s * per_sub + t * k
        pltpu.sync_copy(ia.at[pl.ds(c * e + base, k)], idx_v)
        pltpu.sync_copy(mm.at[c, pl.ds(base, k)], m_v)
        for g in range(k // 16):
            v = idx_v[pl.ds(g * 16, 16)]
            lv = v - (j * half)
            ok = (lv >= 0) & (lv < half)
            idxt_v[pl.ds(g * 16, 16)] = jnp.where(ok, lv, half)
        pltpu.sync_copy(m_v, accm.at[idxt_v], add=True)
        return _

    lax.fori_loop(0, chunks, step, None)
    plsc.subcore_barrier()
    pltpu.sync_copy(accm.at[pl.ds(s * wrows, wrows)],
                    outm.at[c, pl.ds(s * wrows, wrows)])


def _k4x_body(npad, e, k, ww, ia, zx4, outx, idx_v, w_v, accx_t):
    c = lax.axis_index("c")
    s = lax.axis_index("s")
    per_sub = e // NS
    chunks = per_sub // k

    pltpu.sync_copy(zx4, accx_t)

    def step(t, _):
        base = s * per_sub + t * k
        pltpu.sync_copy(ia.at[pl.ds(c * e + base, k)], idx_v)
        for col in range(3):
            pltpu.sync_copy(ww.at[pl.ds((c * 4 + col) * e + base, k)],
                            w_v.at[pl.ds(col * k, k)])
        for g in range(k // 16):
            v = idx_v[pl.ds(g * 16, 16)]
            for col in range(3):
                vals = w_v[pl.ds(col * k + g * 16, 16)]
                plsc.addupdate_scatter(accx_t, [v * 4 + col], vals)
        return _

    lax.fori_loop(0, chunks, step, None)
    pltpu.sync_copy(accx_t, outx.at[c, s])


def _scatter_stage(mm, ww, ia_flat, n, e, d):
    k = 400
    npad = ((n + NS * 16 - 1) // (NS * 16)) * NS * 16
    half = npad // 2
    rr = half + NS * 8
    zm = jnp.zeros((rr, d), jnp.float32)
    zx4 = jnp.zeros((npad * 4,), jnp.float32)
    mesh = plsc.VectorSubcoreMesh(core_axis_name="c", subcore_axis_name="s")

    halves = []
    for j in (0, 1):
        halves.append(pl.kernel(
            functools.partial(_k4m_body, half, rr, e, k, j),
            out_type=jax.ShapeDtypeStruct((2, half, d), jnp.float32),
            mesh=mesh,
            scratch_types=[
                pltpu.VMEM((k,), jnp.int32),
                pltpu.VMEM((k,), jnp.int32),
                pltpu.VMEM((k, d), jnp.float32),
                pltpu.VMEM_SHARED((rr, d), jnp.float32),
            ],
        )(mm, ia_flat, zm))

    outx = pl.kernel(
        functools.partial(_k4x_body, npad, e, k),
        out_type=jax.ShapeDtypeStruct((2, NS, npad * 4), jnp.float32),
        mesh=mesh,
        scratch_types=[
            pltpu.VMEM((k,), jnp.int32),
            pltpu.VMEM((4 * k,), jnp.float32),
            pltpu.VMEM((npad * 4,), jnp.float32),
        ],
        compiler_params=pltpu.CompilerParams(needs_layout_passes=False),
    )(ww.reshape(2 * 4 * e), ia_flat, zx4)

    magg = jnp.concatenate(halves, axis=1)
    return magg, outx.reshape(2 * NS, npad * 4), npad


def _k5h_body(h_ref, ma_ref, w1h_ref, w1u_ref, w1d_ref,
              b1_ref, w2_ref, b2_ref, hout_ref):
    pre = (jnp.dot(h_ref[...], w1h_ref[...], preferred_element_type=jnp.float32)
           + jnp.dot(ma_ref[0], w1u_ref[...], preferred_element_type=jnp.float32)
           + jnp.dot(ma_ref[1], w1d_ref[...], preferred_element_type=jnp.float32)
           + b1_ref[0])
    hout_ref[...] = h_ref[...] + jnp.dot(
        jnp.maximum(pre, 0.0), w2_ref[...],
        preferred_element_type=jnp.float32) + b2_ref[0]


def _node_update_h(h, ma, w1h, w1u, w1d, b1, w2, b2, n, d):
    bn = 2000
    grid = (n // bn,)
    return pl.pallas_call(
        _k5h_body,
        grid=grid,
        in_specs=[
            pl.BlockSpec((bn, d), lambda i: (i, 0)),
            pl.BlockSpec((2, bn, d), lambda i: (0, i, 0)),
            pl.BlockSpec((d, d), lambda i: (0, 0)),
            pl.BlockSpec((d, d), lambda i: (0, 0)),
            pl.BlockSpec((d, d), lambda i: (0, 0)),
            pl.BlockSpec((1, d), lambda i: (0, 0)),
            pl.BlockSpec((d, d), lambda i: (0, 0)),
            pl.BlockSpec((1, d), lambda i: (0, 0)),
        ],
        out_specs=pl.BlockSpec((bn, d), lambda i: (i, 0)),
        out_shape=jax.ShapeDtypeStruct((n, d), jnp.float32),
    )(h, ma, w1h, w1u, w1d, b1, w2, b2)


def _k5x_body(xagg_ref, xpad_ref, xout_ref):
    xout_ref[...] = xpad_ref[...] + jnp.sum(
        xagg_ref[...], axis=0, keepdims=True)


def _node_update_x(xaggf, xpad4f, npad):
    bx = 5120
    grid = (npad * 4 // bx,)
    return pl.pallas_call(
        _k5x_body,
        grid=grid,
        in_specs=[
            pl.BlockSpec((2 * NS, bx), lambda i: (0, i)),
            pl.BlockSpec((1, bx), lambda i: (0, i)),
        ],
        out_specs=pl.BlockSpec((1, bx), lambda i: (0, i)),
        out_shape=jax.ShapeDtypeStruct((1, npad * 4), jnp.float32),
    )(xaggf, xpad4f)


def kernel(h, h_up, h_down, x, x_up, x_down, b_up, b_down, cw,
           bu_w1, bu_b1, bu_w2, bu_b2,
           bd_w1, bd_b1, bd_w2, bd_b2,
           cm_w1, cm_b1, cm_w2, cm_b2,
           cu_w1, cu_b1, cu_w2, cu_b2,
           cd_w1, cd_b1, cd_w2, cd_b2):
    n, d = h.shape
    e = b_up.shape[1]

    hs = jnp.stack([h_up, h_down])
    wa = jnp.stack([bu_w1[:d], bd_w1[:d]])
    wb = jnp.stack([bu_w1[d:2 * d], bd_w1[d:2 * d]])
    w1c = jnp.stack([bu_w1[2 * d], bd_w1[2 * d]]).reshape(2, 1, d)
    b1s = jnp.stack([bu_b1, bd_b1]).reshape(2, 1, d)
    w2b = jnp.stack([bu_w2, bd_w2]).astype(jnp.bfloat16)
    b2s = jnp.stack([bu_b2, bd_b2]).reshape(2, 1, d)
    cw1b = jnp.stack([cu_w1, cd_w1]).astype(jnp.bfloat16)
    cb1s = jnp.stack([cu_b1, cd_b1]).reshape(2, 1, d)
    cw2t = jnp.stack([cu_w2.T, cd_w2.T])
    cb2s = jnp.stack([cu_b2, cd_b2])

    xpad = jnp.pad(x, ((0, 0), (0, XP - 3)))
    xnbs = jnp.stack([jnp.pad(x_up, ((0, 0), (0, XP - 3))),
                      jnp.pad(x_down, ((0, 0), (0, XP - 3)))])

    ia_raw = jnp.stack([b_up[0], b_down[1]])
    ib_raw = jnp.stack([b_up[1], b_down[0]])
    off = jnp.array([[0], [n]], jnp.int32)
    ia_off = (ia_raw + off).reshape(2 * e)
    ib_off = (ib_raw + off).reshape(2 * e)
    ia_flat = ia_raw.reshape(2 * e)

    ta, tb = _make_tables(h, hs, xpad, xnbs, wa, wb, b1s, w1c, n, d)
    tacat = lax.bitcast_convert_type(
        ta.reshape(2 * n, TW // 2, 2), jnp.int32)
    tbcat = lax.bitcast_convert_type(
        tb.reshape(2 * n, TW // 2, 2), jnp.int32)

    ga_p, gb_p = _gather_stage(tacat, tbcat, ia_off, ib_off, e)
    ga = lax.bitcast_convert_type(ga_p, jnp.bfloat16).reshape(2, e, TW)
    gb = lax.bitcast_convert_type(gb_p, jnp.bfloat16).reshape(2, e, TW)

    mm, ww = _edge_mlp(ga, gb, w1c, w2b, b2s, cw1b, cb1s, cw2t,
                       cb2s, cw, e, d)

    magg, xaggf, npad = _scatter_stage(mm, ww, ia_flat, n, e, d)

    hout = _node_update_h(
        h, magg,
        cm_w1[:d], cm_w1[d:2 * d], cm_w1[2 * d:], cm_b1.reshape(1, d),
        cm_w2, cm_b2.reshape(1, d), n, d)
    xpad4f = jnp.pad(x, ((0, npad - n), (0, 1))).reshape(1, npad * 4)
    xoutf = _node_update_x(xaggf, xpad4f, npad)

    return (hout, xoutf.reshape(npad, 4)[:n, :3])


# speedup vs baseline: 5.2575x; 2.8139x over previous
import functools

import jax
import jax.numpy as jnp
from jax import lax
from jax.experimental import pallas as pl
from jax.experimental.pallas import tpu as pltpu
from jax.experimental.pallas import tpu_sc as plsc

NC = 2
NS = 16
XP = 16


def _rne16(f):
    u = lax.bitcast_convert_type(f, jnp.uint32)
    return (u + ((u >> 16) & 1) + 0x7FFF) >> 16


def _pack(feat, xpad16, d):
    xfull = jnp.concatenate(
        [xpad16, jnp.zeros((xpad16.shape[0], d - xpad16.shape[1]),
                           jnp.float32)], axis=1)
    packed = _rne16(feat) | (_rne16(xfull) << 16)
    return lax.bitcast_convert_type(packed, jnp.int32)


def _k1_body(h_ref, hs_ref, xpad_ref, xnbs_ref, wa_ref, wb_ref, b1_ref,
             w1c_ref, ta_ref, tb_ref):
    d = h_ref.shape[1]
    xa = xpad_ref[...]
    xb = xnbs_ref[0]
    x2a = jnp.sum(xa * xa, axis=-1, keepdims=True)
    x2b = jnp.sum(xb * xb, axis=-1, keepdims=True)
    pa = (jnp.dot(h_ref[...], wa_ref[0], preferred_element_type=jnp.float32)
          + b1_ref[0] + x2a * w1c_ref[0])
    qb = (jnp.dot(hs_ref[0], wb_ref[0], preferred_element_type=jnp.float32)
          + x2b * w1c_ref[0])
    ta_ref[0] = _pack(pa, xa, d)
    tb_ref[0] = _pack(qb, xb, d)


def _make_tables(h, hs, xpad, xnbs, wa, wb, b1s, w1c, n, d):
    bn = 2000
    grid = (2, n // bn)
    return pl.pallas_call(
        _k1_body,
        grid=grid,
        in_specs=[
            pl.BlockSpec((bn, d), lambda c, i: (i, 0)),
            pl.BlockSpec((1, bn, d), lambda c, i: (c, i, 0)),
            pl.BlockSpec((bn, XP), lambda c, i: (i, 0)),
            pl.BlockSpec((1, bn, XP), lambda c, i: (c, i, 0)),
            pl.BlockSpec((1, d, d), lambda c, i: (c, 0, 0)),
            pl.BlockSpec((1, d, d), lambda c, i: (c, 0, 0)),
            pl.BlockSpec((1, 1, d), lambda c, i: (c, 0, 0)),
            pl.BlockSpec((1, 1, d), lambda c, i: (c, 0, 0)),
        ],
        out_specs=[
            pl.BlockSpec((1, bn, d), lambda c, i: (c, i, 0)),
            pl.BlockSpec((1, bn, d), lambda c, i: (c, i, 0)),
        ],
        out_shape=[
            jax.ShapeDtypeStruct((2, n, d), jnp.int32),
            jax.ShapeDtypeStruct((2, n, d), jnp.int32),
        ],
    )(h, hs, xpad, xnbs, wa, wb, b1s, w1c)


def _k2_body(e, k, ta, tb, ia, ib, ga, gb,
             idxa_v, idxb_v, bufa, bufb, sem):
    c = lax.axis_index("c")
    s = lax.axis_index("s")
    per_sub = e // NS
    chunks = per_sub // k

    def step(t, _):
        base = s * per_sub + t * k
        pltpu.sync_copy(ia.at[pl.ds(c * e + base, k)], idxa_v)
        pltpu.sync_copy(ib.at[pl.ds(c * e + base, k)], idxb_v)
        cp1 = pltpu.async_copy(ta.at[idxa_v], bufa, sem)
        cp2 = pltpu.async_copy(tb.at[idxb_v], bufb, sem)
        cp1.wait()
        cp2.wait()
        pltpu.sync_copy(bufa, ga.at[c, pl.ds(base, k)])
        pltpu.sync_copy(bufb, gb.at[c, pl.ds(base, k)])
        return _

    lax.fori_loop(0, chunks, step, None)


def _gather_stage(ta, tb, ia_off, ib_off, e, d):
    k = 400
    tw2 = d
    mesh = plsc.VectorSubcoreMesh(core_axis_name="c", subcore_axis_name="s")
    fn = pl.kernel(
        functools.partial(_k2_body, e, k),
        out_type=[
            jax.ShapeDtypeStruct((2, e, tw2), jnp.int32),
            jax.ShapeDtypeStruct((2, e, tw2), jnp.int32),
        ],
        mesh=mesh,
        scratch_types=[
            pltpu.VMEM((k,), jnp.int32),
            pltpu.VMEM((k,), jnp.int32),
            pltpu.VMEM((k, tw2), jnp.int32),
            pltpu.VMEM((k, tw2), jnp.int32),
            pltpu.SemaphoreType.DMA,
        ],
    )
    return fn(ta, tb, ia_off, ib_off)


def _k3_body(d, ga_ref, gb_ref, w1c_ref, w2_ref, b2_ref,
             cw1_ref, cb1_ref, cw2t_ref, cb2_ref, cws_ref, m_ref, w_ref):
    di = pl.program_id(0)
    au = lax.bitcast_convert_type(ga_ref[0], jnp.uint32)
    bu = lax.bitcast_convert_type(gb_ref[0], jnp.uint32)
    ap = lax.bitcast_convert_type(au << 16, jnp.float32)
    bq = lax.bitcast_convert_type(bu << 16, jnp.float32)
    msk = jnp.uint32(0xFFFF0000)
    xa = lax.bitcast_convert_type(au & msk, jnp.float32)[:, :XP]
    xb = lax.bitcast_convert_type(bu & msk, jnp.float32)[:, :XP]
    sij = jnp.sum(xa * xb, axis=-1, keepdims=True)
    xd = xa - xb
    u = jnp.maximum(ap + bq - 2.0 * sij * w1c_ref[0], 0.0)
    m = jnp.dot(u.astype(jnp.bfloat16), w2_ref[0],
                preferred_element_type=jnp.float32) + b2_ref[0]
    g = jnp.maximum(
        jnp.dot(m.astype(jnp.bfloat16), cw1_ref[0],
                preferred_element_type=jnp.float32) + cb1_ref[0], 0.0)
    sp = jnp.sum(g * cw2t_ref[0], axis=-1, keepdims=True) + cb2_ref[di, 0]
    gate = jax.nn.sigmoid(sp)
    m_ref[0] = m
    wv = xd[:, :4] * (gate * cws_ref[di])
    w_ref[0] = wv.T


def _edge_mlp(ga, gb, w1c, w2b, b2s, cw1b, cb1s, cw2t, cb2s, cws, e, d):
    kb = 2560
    grid = (2, e // kb)
    return pl.pallas_call(
        functools.partial(_k3_body, d),
        grid=grid,
        in_specs=[
            pl.BlockSpec((1, kb, d), lambda c, i: (c, i, 0)),
            pl.BlockSpec((1, kb, d), lambda c, i: (c, i, 0)),
            pl.BlockSpec((1, 1, d), lambda c, i: (c, 0, 0)),
            pl.BlockSpec((1, d, d), lambda c, i: (c, 0, 0)),
            pl.BlockSpec((1, 1, d), lambda c, i: (c, 0, 0)),
            pl.BlockSpec((1, d, d), lambda c, i: (c, 0, 0)),
            pl.BlockSpec((1, 1, d), lambda c, i: (c, 0, 0)),
            pl.BlockSpec((1, 1, d), lambda c, i: (c, 0, 0)),
            pl.BlockSpec(memory_space=pltpu.SMEM),
            pl.BlockSpec(memory_space=pltpu.SMEM),
        ],
        out_specs=[
            pl.BlockSpec((1, kb, d), lambda c, i: (c, i, 0)),
            pl.BlockSpec((1, 4, kb), lambda c, i: (c, 0, i)),
        ],
        out_shape=[
            jax.ShapeDtypeStruct((2, e, d), jnp.float32),
            jax.ShapeDtypeStruct((2, 4, e), jnp.float32),
        ],
    )(ga, gb, w1c, w2b, b2s, cw1b, cb1s, cw2t, cb2s, cws)


def _k4m_body(half, rr, e, k, j, mm, ia, zm, outm,
              idx_v, idxt_v, m_v, accm):
    c = lax.axis_index("c")
    s = lax.axis_index("s")
    per_sub = e // NS
    chunks = per_sub // k
    zrows = rr // NS
    wrows = half // NS

    pltpu.sync_copy(zm.at[pl.ds(s * zrows, zrows)],
                    accm.at[pl.ds(s * zrows, zrows)])
    plsc.subcore_barrier()

    def step(t, _):
        base = s * per_sub + t * k
        pltpu.sync_copy(ia.at[pl.ds(c * e + base, k)], idx_v)
        pltpu.sync_copy(mm.at[c, pl.ds(base, k)], m_v)
        for g in range(k // 16):
            v = idx_v[pl.ds(g * 16, 16)]
            lv = v - (j * half)
            ok = (lv >= 0) & (lv < half)
            idxt_v[pl.ds(g * 16, 16)] = jnp.where(ok, lv, half)
        pltpu.sync_copy(m_v, accm.at[idxt_v], add=True)
        return _

    lax.fori_loop(0, chunks, step, None)
    plsc.subcore_barrier()
    pltpu.sync_copy(accm.at[pl.ds(s * wrows, wrows)],
                    outm.at[c, pl.ds(s * wrows, wrows)])


def _k4x_body(npad, e, k, ww, ia, zx4, outx, idx_v, w_v, accx_t):
    c = lax.axis_index("c")
    s = lax.axis_index("s")
    per_sub = e // NS
    chunks = per_sub // k

    pltpu.sync_copy(zx4, accx_t)

    def step(t, _):
        base = s * per_sub + t * k
        pltpu.sync_copy(ia.at[pl.ds(c * e + base, k)], idx_v)
        for col in range(3):
            pltpu.sync_copy(ww.at[pl.ds((c * 4 + col) * e + base, k)],
                            w_v.at[pl.ds(col * k, k)])
        for g in range(k // 16):
            v = idx_v[pl.ds(g * 16, 16)]
            for col in range(3):
                vals = w_v[pl.ds(col * k + g * 16, 16)]
                plsc.addupdate_scatter(accx_t, [v * 4 + col], vals)
        return _

    lax.fori_loop(0, chunks, step, None)
    pltpu.sync_copy(accx_t, outx.at[c, s])


def _scatter_stage(mm, ww, ia_flat, n, e, d):
    k = 400
    npad = ((n + NS * 16 - 1) // (NS * 16)) * NS * 16
    half = npad // 2
    rr = half + NS * 8
    zm = jnp.zeros((rr, d), jnp.float32)
    zx4 = jnp.zeros((npad * 4,), jnp.float32)
    mesh = plsc.VectorSubcoreMesh(core_axis_name="c", subcore_axis_name="s")

    halves = []
    for j in (0, 1):
        halves.append(pl.kernel(
            functools.partial(_k4m_body, half, rr, e, k, j),
            out_type=jax.ShapeDtypeStruct((2, half, d), jnp.float32),
            mesh=mesh,
            scratch_types=[
                pltpu.VMEM((k,), jnp.int32),
                pltpu.VMEM((k,), jnp.int32),
                pltpu.VMEM((k, d), jnp.float32),
                pltpu.VMEM_SHARED((rr, d), jnp.float32),
            ],
        )(mm, ia_flat, zm))

    outx = pl.kernel(
        functools.partial(_k4x_body, npad, e, k),
        out_type=jax.ShapeDtypeStruct((2, NS, npad * 4), jnp.float32),
        mesh=mesh,
        scratch_types=[
            pltpu.VMEM((k,), jnp.int32),
            pltpu.VMEM((4 * k,), jnp.float32),
            pltpu.VMEM((npad * 4,), jnp.float32),
        ],
        compiler_params=pltpu.CompilerParams(needs_layout_passes=False),
    )(ww.reshape(2 * 4 * e), ia_flat, zx4)

    magg = jnp.concatenate(halves, axis=1)
    return magg, outx.reshape(2 * NS, npad * 4), npad


def _k5h_body(h_ref, ma_ref, w1h_ref, w1u_ref, w1d_ref,
              b1_ref, w2_ref, b2_ref, hout_ref):
    pre = (jnp.dot(h_ref[...], w1h_ref[...], preferred_element_type=jnp.float32)
           + jnp.dot(ma_ref[0], w1u_ref[...], preferred_element_type=jnp.float32)
           + jnp.dot(ma_ref[1], w1d_ref[...], preferred_element_type=jnp.float32)
           + b1_ref[0])
    hout_ref[...] = h_ref[...] + jnp.dot(
        jnp.maximum(pre, 0.0), w2_ref[...],
        preferred_element_type=jnp.float32) + b2_ref[0]


def _node_update_h(h, ma, w1h, w1u, w1d, b1, w2, b2, n, d):
    bn = 2000
    grid = (n // bn,)
    return pl.pallas_call(
        _k5h_body,
        grid=grid,
        in_specs=[
            pl.BlockSpec((bn, d), lambda i: (i, 0)),
            pl.BlockSpec((2, bn, d), lambda i: (0, i, 0)),
            pl.BlockSpec((d, d), lambda i: (0, 0)),
            pl.BlockSpec((d, d), lambda i: (0, 0)),
            pl.BlockSpec((d, d), lambda i: (0, 0)),
            pl.BlockSpec((1, d), lambda i: (0, 0)),
            pl.BlockSpec((d, d), lambda i: (0, 0)),
            pl.BlockSpec((1, d), lambda i: (0, 0)),
        ],
        out_specs=pl.BlockSpec((bn, d), lambda i: (i, 0)),
        out_shape=jax.ShapeDtypeStruct((n, d), jnp.float32),
    )(h, ma, w1h, w1u, w1d, b1, w2, b2)


def _k5x_body(xagg_ref, xpad_ref, xout_ref):
    xout_ref[...] = xpad_ref[...] + jnp.sum(
        xagg_ref[...], axis=0, keepdims=True)


def _node_update_x(xaggf, xpad4f, npad):
    bx = 5120
    grid = (npad * 4 // bx,)
    return pl.pallas_call(
        _k5x_body,
        grid=grid,
        in_specs=[
            pl.BlockSpec((2 * NS, bx), lambda i: (0, i)),
            pl.BlockSpec((1, bx), lambda i: (0, i)),
        ],
        out_specs=pl.BlockSpec((1, bx), lambda i: (0, i)),
        out_shape=jax.ShapeDtypeStruct((1, npad * 4), jnp.float32),
    )(xaggf, xpad4f)


def kernel(h, h_up, h_down, x, x_up, x_down, b_up, b_down, cw,
           bu_w1, bu_b1, bu_w2, bu_b2,
           bd_w1, bd_b1, bd_w2, bd_b2,
           cm_w1, cm_b1, cm_w2, cm_b2,
           cu_w1, cu_b1, cu_w2, cu_b2,
           cd_w1, cd_b1, cd_w2, cd_b2):
    n, d = h.shape
    e = b_up.shape[1]

    hs = jnp.stack([h_up, h_down])
    wa = jnp.stack([bu_w1[:d], bd_w1[:d]])
    wb = jnp.stack([bu_w1[d:2 * d], bd_w1[d:2 * d]])
    w1c = jnp.stack([bu_w1[2 * d], bd_w1[2 * d]]).reshape(2, 1, d)
    b1s = jnp.stack([bu_b1, bd_b1]).reshape(2, 1, d)
    w2b = jnp.stack([bu_w2, bd_w2]).astype(jnp.bfloat16)
    b2s = jnp.stack([bu_b2, bd_b2]).reshape(2, 1, d)
    cw1b = jnp.stack([cu_w1, cd_w1]).astype(jnp.bfloat16)
    cb1s = jnp.stack([cu_b1, cd_b1]).reshape(2, 1, d)
    cw2t = jnp.stack([cu_w2.T, cd_w2.T])
    cb2s = jnp.stack([cu_b2, cd_b2])

    xpad = jnp.pad(x, ((0, 0), (0, XP - 3)))
    xnbs = jnp.stack([jnp.pad(x_up, ((0, 0), (0, XP - 3))),
                      jnp.pad(x_down, ((0, 0), (0, XP - 3)))])

    ia_raw = jnp.stack([b_up[0], b_down[1]])
    ib_raw = jnp.stack([b_up[1], b_down[0]])
    off = jnp.array([[0], [n]], jnp.int32)
    ia_off = (ia_raw + off).reshape(2 * e)
    ib_off = (ib_raw + off).reshape(2 * e)
    ia_flat = ia_raw.reshape(2 * e)

    ta, tb = _make_tables(h, hs, xpad, xnbs, wa, wb, b1s, w1c, n, d)
    tacat = ta.reshape(2 * n, d)
    tbcat = tb.reshape(2 * n, d)

    ga, gb = _gather_stage(tacat, tbcat, ia_off, ib_off, e, d)

    mm, ww = _edge_mlp(ga, gb, w1c, w2b, b2s, cw1b, cb1s, cw2t,
                       cb2s, cw, e, d)

    magg, xaggf, npad = _scatter_stage(mm, ww, ia_flat, n, e, d)

    hout = _node_update_h(
        h, magg,
        cm_w1[:d], cm_w1[d:2 * d], cm_w1[2 * d:], cm_b1.reshape(1, d),
        cm_w2, cm_b2.reshape(1, d), n, d)
    xpad4f = jnp.pad(x, ((0, npad - n), (0, 1))).reshape(1, npad * 4)
    xoutf = _node_update_x(xaggf, xpad4f, npad)

    return (hout, xoutf.reshape(npad, 4)[:n, :3])


# speedup vs baseline: 5.5616x; 1.0578x over previous
import functools

import jax
import jax.numpy as jnp
from jax import lax
from jax.experimental import pallas as pl
from jax.experimental.pallas import tpu as pltpu
from jax.experimental.pallas import tpu_sc as plsc

NC = 2
NS = 16
XP = 16


def _rne16(f):
    u = lax.bitcast_convert_type(f, jnp.uint32)
    return (u + ((u >> 16) & 1) + 0x7FFF) >> 16


def _pack(feat, xpad16, d):
    xfull = jnp.concatenate(
        [xpad16, jnp.zeros((xpad16.shape[0], d - xpad16.shape[1]),
                           jnp.float32)], axis=1)
    packed = _rne16(feat) | (_rne16(xfull) << 16)
    return lax.bitcast_convert_type(packed, jnp.int32)


def _k1_body(h_ref, hs_ref, xpad_ref, xnbs_ref, wa_ref, wb_ref, b1_ref,
             w1c_ref, ta_ref, tb_ref):
    d = h_ref.shape[1]
    xa = xpad_ref[...]
    xb = xnbs_ref[0]
    x2a = jnp.sum(xa * xa, axis=-1, keepdims=True)
    x2b = jnp.sum(xb * xb, axis=-1, keepdims=True)
    pa = (jnp.dot(h_ref[...], wa_ref[0], preferred_element_type=jnp.float32)
          + b1_ref[0] + x2a * w1c_ref[0])
    qb = (jnp.dot(hs_ref[0], wb_ref[0], preferred_element_type=jnp.float32)
          + x2b * w1c_ref[0])
    ta_ref[0] = _pack(pa, xa, d)
    tb_ref[0] = _pack(qb, xb, d)


def _make_tables(h, hs, xpad, xnbs, wa, wb, b1s, w1c, n, d):
    bn = 2000
    grid = (2, n // bn)
    return pl.pallas_call(
        _k1_body,
        grid=grid,
        in_specs=[
            pl.BlockSpec((bn, d), lambda c, i: (i, 0)),
            pl.BlockSpec((1, bn, d), lambda c, i: (c, i, 0)),
            pl.BlockSpec((bn, XP), lambda c, i: (i, 0)),
            pl.BlockSpec((1, bn, XP), lambda c, i: (c, i, 0)),
            pl.BlockSpec((1, d, d), lambda c, i: (c, 0, 0)),
            pl.BlockSpec((1, d, d), lambda c, i: (c, 0, 0)),
            pl.BlockSpec((1, 1, d), lambda c, i: (c, 0, 0)),
            pl.BlockSpec((1, 1, d), lambda c, i: (c, 0, 0)),
        ],
        out_specs=[
            pl.BlockSpec((1, bn, d), lambda c, i: (c, i, 0)),
            pl.BlockSpec((1, bn, d), lambda c, i: (c, i, 0)),
        ],
        out_shape=[
            jax.ShapeDtypeStruct((2, n, d), jnp.int32),
            jax.ShapeDtypeStruct((2, n, d), jnp.int32),
        ],
    )(h, hs, xpad, xnbs, wa, wb, b1s, w1c)


def _k2_body(e, eh, c0, k, ta, tb, ia, ib, ga, gb,
             idxa_v, idxb_v, bufa, bufb, sem):
    c = lax.axis_index("c")
    s = lax.axis_index("s")
    per_sub = eh // NS
    chunks = per_sub // k

    def step(t, _):
        base = s * per_sub + t * k
        pltpu.sync_copy(ia.at[pl.ds(c * e + c0 + base, k)], idxa_v)
        pltpu.sync_copy(ib.at[pl.ds(c * e + c0 + base, k)], idxb_v)
        cp1 = pltpu.async_copy(ta.at[idxa_v], bufa, sem)
        cp2 = pltpu.async_copy(tb.at[idxb_v], bufb, sem)
        cp1.wait()
        cp2.wait()
        pltpu.sync_copy(bufa, ga.at[c, pl.ds(base, k)])
        pltpu.sync_copy(bufb, gb.at[c, pl.ds(base, k)])
        return _

    lax.fori_loop(0, chunks, step, None)


def _gather_stage(ta, tb, ia_off, ib_off, e, eh, c0, d):
    k = 400
    tw2 = d
    mesh = plsc.VectorSubcoreMesh(core_axis_name="c", subcore_axis_name="s")
    fn = pl.kernel(
        functools.partial(_k2_body, e, eh, c0, k),
        out_type=[
            jax.ShapeDtypeStruct((2, eh, tw2), jnp.int32),
            jax.ShapeDtypeStruct((2, eh, tw2), jnp.int32),
        ],
        mesh=mesh,
        scratch_types=[
            pltpu.VMEM((k,), jnp.int32),
            pltpu.VMEM((k,), jnp.int32),
            pltpu.VMEM((k, tw2), jnp.int32),
            pltpu.VMEM((k, tw2), jnp.int32),
            pltpu.SemaphoreType.DMA,
        ],
    )
    return fn(ta, tb, ia_off, ib_off)


def _k3_body(d, ga_ref, gb_ref, w1c_ref, w2_ref, b2_ref,
             cw1_ref, cb1_ref, cw2t_ref, cb2_ref, cws_ref, m_ref, w_ref):
    di = pl.program_id(0)
    au = lax.bitcast_convert_type(ga_ref[0], jnp.uint32)
    bu = lax.bitcast_convert_type(gb_ref[0], jnp.uint32)
    ap = lax.bitcast_convert_type(au << 16, jnp.float32)
    bq = lax.bitcast_convert_type(bu << 16, jnp.float32)
    msk = jnp.uint32(0xFFFF0000)
    xa = lax.bitcast_convert_type(au & msk, jnp.float32)[:, :XP]
    xb = lax.bitcast_convert_type(bu & msk, jnp.float32)[:, :XP]
    sij = jnp.sum(xa * xb, axis=-1, keepdims=True)
    xd = xa - xb
    u = jnp.maximum(ap + bq - 2.0 * sij * w1c_ref[0], 0.0)
    m = jnp.dot(u.astype(jnp.bfloat16), w2_ref[0],
                preferred_element_type=jnp.float32) + b2_ref[0]
    g = jnp.maximum(
        jnp.dot(m.astype(jnp.bfloat16), cw1_ref[0],
                preferred_element_type=jnp.float32) + cb1_ref[0], 0.0)
    sp = jnp.sum(g * cw2t_ref[0], axis=-1, keepdims=True) + cb2_ref[di, 0]
    gate = jax.nn.sigmoid(sp)
    m_ref[0] = m
    wv = xd[:, :4] * (gate * cws_ref[di])
    w_ref[0] = wv.T


def _edge_mlp(ga, gb, w1c, w2b, b2s, cw1b, cb1s, cw2t, cb2s, cws, e, d):
    kb = 3200
    grid = (2, e // kb)
    return pl.pallas_call(
        functools.partial(_k3_body, d),
        grid=grid,
        in_specs=[
            pl.BlockSpec((1, kb, d), lambda c, i: (c, i, 0)),
            pl.BlockSpec((1, kb, d), lambda c, i: (c, i, 0)),
            pl.BlockSpec((1, 1, d), lambda c, i: (c, 0, 0)),
            pl.BlockSpec((1, d, d), lambda c, i: (c, 0, 0)),
            pl.BlockSpec((1, 1, d), lambda c, i: (c, 0, 0)),
            pl.BlockSpec((1, d, d), lambda c, i: (c, 0, 0)),
            pl.BlockSpec((1, 1, d), lambda c, i: (c, 0, 0)),
            pl.BlockSpec((1, 1, d), lambda c, i: (c, 0, 0)),
            pl.BlockSpec(memory_space=pltpu.SMEM),
            pl.BlockSpec(memory_space=pltpu.SMEM),
        ],
        out_specs=[
            pl.BlockSpec((1, kb, d), lambda c, i: (c, i, 0)),
            pl.BlockSpec((1, 4, kb), lambda c, i: (c, 0, i)),
        ],
        out_shape=[
            jax.ShapeDtypeStruct((2, e, d), jnp.float32),
            jax.ShapeDtypeStruct((2, 4, e), jnp.float32),
        ],
    )(ga, gb, w1c, w2b, b2s, cw1b, cb1s, cw2t, cb2s, cws)


def _k4m_body(half, rr, e, eh, k, j, mm0, mm1, ia, zm, outm,
              idx_v, idxt_v, m_v, accm):
    c = lax.axis_index("c")
    s = lax.axis_index("s")
    per_sub = eh // NS
    chunks = per_sub // k
    zrows = rr // NS
    wrows = half // NS

    pltpu.sync_copy(zm.at[pl.ds(s * zrows, zrows)],
                    accm.at[pl.ds(s * zrows, zrows)])
    plsc.subcore_barrier()

    def step(t, _):
        base = s * per_sub + t * k
        for ci, mm in ((0, mm0), (1, mm1)):
            pltpu.sync_copy(ia.at[pl.ds(c * e + ci * eh + base, k)], idx_v)
            pltpu.sync_copy(mm.at[c, pl.ds(base, k)], m_v)
            for g in range(k // 16):
                v = idx_v[pl.ds(g * 16, 16)]
                lv = v - (j * half)
                ok = (lv >= 0) & (lv < half)
                idxt_v[pl.ds(g * 16, 16)] = jnp.where(ok, lv, half)
            pltpu.sync_copy(m_v, accm.at[idxt_v], add=True)
        return _

    lax.fori_loop(0, chunks, step, None)
    plsc.subcore_barrier()
    pltpu.sync_copy(accm.at[pl.ds(s * wrows, wrows)],
                    outm.at[c, pl.ds(s * wrows, wrows)])


def _k4x_body(npad, e, eh, k, ww0, ww1, ia, zx4, outx, idx_v, w_v, accx_t):
    c = lax.axis_index("c")
    s = lax.axis_index("s")
    per_sub = eh // NS
    chunks = per_sub // k

    pltpu.sync_copy(zx4, accx_t)

    def step(t, _):
        base = s * per_sub + t * k
        for ci, ww in ((0, ww0), (1, ww1)):
            pltpu.sync_copy(ia.at[pl.ds(c * e + ci * eh + base, k)], idx_v)
            for col in range(3):
                pltpu.sync_copy(ww.at[pl.ds((c * 4 + col) * eh + base, k)],
                                w_v.at[pl.ds(col * k, k)])
            for g in range(k // 16):
                v = idx_v[pl.ds(g * 16, 16)]
                for col in range(3):
                    vals = w_v[pl.ds(col * k + g * 16, 16)]
                    plsc.addupdate_scatter(accx_t, [v * 4 + col], vals)
        return _

    lax.fori_loop(0, chunks, step, None)
    pltpu.sync_copy(accx_t, outx.at[c, s])


def _scatter_stage(mms, wws, ia_flat, n, e, eh, d):
    k = 400
    npad = ((n + NS * 16 - 1) // (NS * 16)) * NS * 16
    half = npad // 2
    rr = half + NS * 8
    zm = jnp.zeros((rr, d), jnp.float32)
    zx4 = jnp.zeros((npad * 4,), jnp.float32)
    mesh = plsc.VectorSubcoreMesh(core_axis_name="c", subcore_axis_name="s")

    halves = []
    for j in (0, 1):
        halves.append(pl.kernel(
            functools.partial(_k4m_body, half, rr, e, eh, k, j),
            out_type=jax.ShapeDtypeStruct((2, half, d), jnp.float32),
            mesh=mesh,
            scratch_types=[
                pltpu.VMEM((k,), jnp.int32),
                pltpu.VMEM((k,), jnp.int32),
                pltpu.VMEM((k, d), jnp.float32),
                pltpu.VMEM_SHARED((rr, d), jnp.float32),
            ],
        )(mms[0], mms[1], ia_flat, zm))

    outx = pl.kernel(
        functools.partial(_k4x_body, npad, e, eh, k),
        out_type=jax.ShapeDtypeStruct((2, NS, npad * 4), jnp.float32),
        mesh=mesh,
        scratch_types=[
            pltpu.VMEM((k,), jnp.int32),
            pltpu.VMEM((4 * k,), jnp.float32),
            pltpu.VMEM((npad * 4,), jnp.float32),
        ],
        compiler_params=pltpu.CompilerParams(needs_layout_passes=False),
    )(wws[0].reshape(2 * 4 * eh), wws[1].reshape(2 * 4 * eh), ia_flat, zx4)

    magg = jnp.concatenate(halves, axis=1)
    return magg, outx.reshape(2 * NS, npad * 4), npad


def _k5h_body(h_ref, ma_ref, w1h_ref, w1u_ref, w1d_ref,
              b1_ref, w2_ref, b2_ref, hout_ref):
    pre = (jnp.dot(h_ref[...], w1h_ref[...], preferred_element_type=jnp.float32)
           + jnp.dot(ma_ref[0], w1u_ref[...], preferred_element_type=jnp.float32)
           + jnp.dot(ma_ref[1], w1d_ref[...], preferred_element_type=jnp.float32)
           + b1_ref[0])
    hout_ref[...] = h_ref[...] + jnp.dot(
        jnp.maximum(pre, 0.0), w2_ref[...],
        preferred_element_type=jnp.float32) + b2_ref[0]


def _node_update_h(h, ma, w1h, w1u, w1d, b1, w2, b2, n, d):
    bn = 2000
    grid = (n // bn,)
    return pl.pallas_call(
        _k5h_body,
        grid=grid,
        in_specs=[
            pl.BlockSpec((bn, d), lambda i: (i, 0)),
            pl.BlockSpec((2, bn, d), lambda i: (0, i, 0)),
            pl.BlockSpec((d, d), lambda i: (0, 0)),
            pl.BlockSpec((d, d), lambda i: (0, 0)),
            pl.BlockSpec((d, d), lambda i: (0, 0)),
            pl.BlockSpec((1, d), lambda i: (0, 0)),
            pl.BlockSpec((d, d), lambda i: (0, 0)),
            pl.BlockSpec((1, d), lambda i: (0, 0)),
        ],
        out_specs=pl.BlockSpec((bn, d), lambda i: (i, 0)),
        out_shape=jax.ShapeDtypeStruct((n, d), jnp.float32),
    )(h, ma, w1h, w1u, w1d, b1, w2, b2)


def _k5x_body(xagg_ref, xpad_ref, xout_ref):
    xout_ref[...] = xpad_ref[...] + jnp.sum(
        xagg_ref[...], axis=0, keepdims=True)


def _node_update_x(xaggf, xpad4f, npad):
    bx = 5120
    grid = (npad * 4 // bx,)
    return pl.pallas_call(
        _k5x_body,
        grid=grid,
        in_specs=[
            pl.BlockSpec((2 * NS, bx), lambda i: (0, i)),
            pl.BlockSpec((1, bx), lambda i: (0, i)),
        ],
        out_specs=pl.BlockSpec((1, bx), lambda i: (0, i)),
        out_shape=jax.ShapeDtypeStruct((1, npad * 4), jnp.float32),
    )(xaggf, xpad4f)


def kernel(h, h_up, h_down, x, x_up, x_down, b_up, b_down, cw,
           bu_w1, bu_b1, bu_w2, bu_b2,
           bd_w1, bd_b1, bd_w2, bd_b2,
           cm_w1, cm_b1, cm_w2, cm_b2,
           cu_w1, cu_b1, cu_w2, cu_b2,
           cd_w1, cd_b1, cd_w2, cd_b2):
    n, d = h.shape
    e = b_up.shape[1]

    hs = jnp.stack([h_up, h_down])
    wa = jnp.stack([bu_w1[:d], bd_w1[:d]])
    wb = jnp.stack([bu_w1[d:2 * d], bd_w1[d:2 * d]])
    w1c = jnp.stack([bu_w1[2 * d], bd_w1[2 * d]]).reshape(2, 1, d)
    b1s = jnp.stack([bu_b1, bd_b1]).reshape(2, 1, d)
    w2b = jnp.stack([bu_w2, bd_w2]).astype(jnp.bfloat16)
    b2s = jnp.stack([bu_b2, bd_b2]).reshape(2, 1, d)
    cw1b = jnp.stack([cu_w1, cd_w1]).astype(jnp.bfloat16)
    cb1s = jnp.stack([cu_b1, cd_b1]).reshape(2, 1, d)
    cw2t = jnp.stack([cu_w2.T, cd_w2.T])
    cb2s = jnp.stack([cu_b2, cd_b2])

    xpad = jnp.pad(x, ((0, 0), (0, XP - 3)))
    xnbs = jnp.stack([jnp.pad(x_up, ((0, 0), (0, XP - 3))),
                      jnp.pad(x_down, ((0, 0), (0, XP - 3)))])

    ia_raw = jnp.stack([b_up[0], b_down[1]])
    ib_raw = jnp.stack([b_up[1], b_down[0]])
    off = jnp.array([[0], [n]], jnp.int32)
    ia_off = (ia_raw + off).reshape(2 * e)
    ib_off = (ib_raw + off).reshape(2 * e)
    ia_flat = ia_raw.reshape(2 * e)

    ta, tb = _make_tables(h, hs, xpad, xnbs, wa, wb, b1s, w1c, n, d)
    tacat = ta.reshape(2 * n, d)
    tbcat = tb.reshape(2 * n, d)

    eh = e // 2
    mms, wws = [], []
    for ci in (0, 1):
        ga, gb = _gather_stage(tacat, tbcat, ia_off, ib_off, e, eh,
                               ci * eh, d)
        mm, ww = _edge_mlp(ga, gb, w1c, w2b, b2s, cw1b, cb1s, cw2t,
                           cb2s, cw, eh, d)
        mms.append(mm)
        wws.append(ww)

    magg, xaggf, npad = _scatter_stage(mms, wws, ia_flat, n, e, eh, d)

    hout = _node_update_h(
        h, magg,
        cm_w1[:d], cm_w1[d:2 * d], cm_w1[2 * d:], cm_b1.reshape(1, d),
        cm_w2, cm_b2.reshape(1, d), n, d)
    xpad4f = jnp.pad(x, ((0, npad - n), (0, 1))).reshape(1, npad * 4)
    xoutf = _node_update_x(xaggf, xpad4f, npad)

    return (hout, xoutf.reshape(npad, 4)[:n, :3])


# speedup vs baseline: 5.5799x; 1.0033x over previous
import functools

import jax
import jax.numpy as jnp
from jax import lax
from jax.experimental import pallas as pl
from jax.experimental.pallas import tpu as pltpu
from jax.experimental.pallas import tpu_sc as plsc

NC = 2
NS = 16
XP = 16


def _rne16(f):
    u = lax.bitcast_convert_type(f, jnp.uint32)
    return (u + ((u >> 16) & 1) + 0x7FFF) >> 16


def _pack(feat, xpad16, d):
    xfull = jnp.concatenate(
        [xpad16, jnp.zeros((xpad16.shape[0], d - xpad16.shape[1]),
                           jnp.float32)], axis=1)
    packed = _rne16(feat) | (_rne16(xfull) << 16)
    return lax.bitcast_convert_type(packed, jnp.int32)


def _k1_body(h_ref, hs_ref, xpad_ref, xnbs_ref, wa_ref, wb_ref, b1_ref,
             w1c_ref, ta_ref, tb_ref):
    d = h_ref.shape[1]
    xa = xpad_ref[...]
    xb = xnbs_ref[0]
    x2a = jnp.sum(xa * xa, axis=-1, keepdims=True)
    x2b = jnp.sum(xb * xb, axis=-1, keepdims=True)
    pa = (jnp.dot(h_ref[...], wa_ref[0], preferred_element_type=jnp.float32)
          + b1_ref[0] + x2a * w1c_ref[0])
    qb = (jnp.dot(hs_ref[0], wb_ref[0], preferred_element_type=jnp.float32)
          + x2b * w1c_ref[0])
    ta_ref[0] = _pack(pa, xa, d)
    tb_ref[0] = _pack(qb, xb, d)


def _make_tables(h, hs, xpad, xnbs, wa, wb, b1s, w1c, n, d):
    bn = 2000
    grid = (2, n // bn)
    return pl.pallas_call(
        _k1_body,
        grid=grid,
        in_specs=[
            pl.BlockSpec((bn, d), lambda c, i: (i, 0)),
            pl.BlockSpec((1, bn, d), lambda c, i: (c, i, 0)),
            pl.BlockSpec((bn, XP), lambda c, i: (i, 0)),
            pl.BlockSpec((1, bn, XP), lambda c, i: (c, i, 0)),
            pl.BlockSpec((1, d, d), lambda c, i: (c, 0, 0)),
            pl.BlockSpec((1, d, d), lambda c, i: (c, 0, 0)),
            pl.BlockSpec((1, 1, d), lambda c, i: (c, 0, 0)),
            pl.BlockSpec((1, 1, d), lambda c, i: (c, 0, 0)),
        ],
        out_specs=[
            pl.BlockSpec((1, bn, d), lambda c, i: (c, i, 0)),
            pl.BlockSpec((1, bn, d), lambda c, i: (c, i, 0)),
        ],
        out_shape=[
            jax.ShapeDtypeStruct((2, n, d), jnp.int32),
            jax.ShapeDtypeStruct((2, n, d), jnp.int32),
        ],
    )(h, hs, xpad, xnbs, wa, wb, b1s, w1c)


def _k2_body(e, eh, c0, k, ta, tb, ia, ib, ga, gb,
             idxa0, idxb0, idxa1, idxb1, bufa0, bufb0, bufa1, bufb1,
             sem0, sem1):
    c = lax.axis_index("c")
    s = lax.axis_index("s")
    per_sub = eh // NS
    chunks = per_sub // k
    assert chunks % 2 == 0 and chunks >= 4
    sets = ((idxa0, idxb0, bufa0, bufb0, sem0),
            (idxa1, idxb1, bufa1, bufb1, sem1))

    def fire(t, st):
        idxa, idxb, bufa, bufb, sem = st
        base = s * per_sub + t * k
        pltpu.sync_copy(ia.at[pl.ds(c * e + c0 + base, k)], idxa)
        pltpu.sync_copy(ib.at[pl.ds(c * e + c0 + base, k)], idxb)
        pltpu.async_copy(ta.at[idxa], bufa, sem)
        pltpu.async_copy(tb.at[idxb], bufb, sem)

    def drain(t, st):
        idxa, idxb, bufa, bufb, sem = st
        base = s * per_sub + t * k
        pltpu.make_async_copy(ta.at[idxa], bufa, sem).wait()
        pltpu.make_async_copy(tb.at[idxb], bufb, sem).wait()
        pltpu.sync_copy(bufa, ga.at[c, pl.ds(base, k)])
        pltpu.sync_copy(bufb, gb.at[c, pl.ds(base, k)])

    fire(0, sets[0])

    def step(t2, _):
        fire(2 * t2 + 1, sets[1])
        drain(2 * t2, sets[0])
        fire(2 * t2 + 2, sets[0])
        drain(2 * t2 + 1, sets[1])
        return _

    lax.fori_loop(0, chunks // 2 - 1, step, None)
    fire(chunks - 1, sets[1])
    drain(chunks - 2, sets[0])
    drain(chunks - 1, sets[1])


def _gather_stage(ta, tb, ia_off, ib_off, e, eh, c0, d):
    k = 200
    tw2 = d
    mesh = plsc.VectorSubcoreMesh(core_axis_name="c", subcore_axis_name="s")
    fn = pl.kernel(
        functools.partial(_k2_body, e, eh, c0, k),
        out_type=[
            jax.ShapeDtypeStruct((2, eh, tw2), jnp.int32),
            jax.ShapeDtypeStruct((2, eh, tw2), jnp.int32),
        ],
        mesh=mesh,
        scratch_types=[
            pltpu.VMEM((k,), jnp.int32),
            pltpu.VMEM((k,), jnp.int32),
            pltpu.VMEM((k,), jnp.int32),
            pltpu.VMEM((k,), jnp.int32),
            pltpu.VMEM((k, tw2), jnp.int32),
            pltpu.VMEM((k, tw2), jnp.int32),
            pltpu.VMEM((k, tw2), jnp.int32),
            pltpu.VMEM((k, tw2), jnp.int32),
            pltpu.SemaphoreType.DMA,
            pltpu.SemaphoreType.DMA,
        ],
    )
    return fn(ta, tb, ia_off, ib_off)


def _k3_body(d, ga_ref, gb_ref, w1c_ref, w2_ref, b2_ref,
             cw1_ref, cb1_ref, cw2t_ref, cb2_ref, cws_ref, m_ref, w_ref):
    di = pl.program_id(0)
    au = lax.bitcast_convert_type(ga_ref[0], jnp.uint32)
    bu = lax.bitcast_convert_type(gb_ref[0], jnp.uint32)
    ap = lax.bitcast_convert_type(au << 16, jnp.float32)
    bq = lax.bitcast_convert_type(bu << 16, jnp.float32)
    msk = jnp.uint32(0xFFFF0000)
    xa = lax.bitcast_convert_type(au & msk, jnp.float32)[:, :XP]
    xb = lax.bitcast_convert_type(bu & msk, jnp.float32)[:, :XP]
    sij = jnp.sum(xa * xb, axis=-1, keepdims=True)
    xd = xa - xb
    u = jnp.maximum(ap + bq - 2.0 * sij * w1c_ref[0], 0.0)
    m = jnp.dot(u.astype(jnp.bfloat16), w2_ref[0],
                preferred_element_type=jnp.float32) + b2_ref[0]
    g = jnp.maximum(
        jnp.dot(m.astype(jnp.bfloat16), cw1_ref[0],
                preferred_element_type=jnp.float32) + cb1_ref[0], 0.0)
    sp = jnp.sum(g * cw2t_ref[0], axis=-1, keepdims=True) + cb2_ref[di, 0]
    gate = jax.nn.sigmoid(sp)
    m_ref[0] = m
    wv = xd[:, :4] * (gate * cws_ref[di])
    w_ref[0] = wv.T


def _edge_mlp(ga, gb, w1c, w2b, b2s, cw1b, cb1s, cw2t, cb2s, cws, e, d):
    kb = 3200
    grid = (2, e // kb)
    return pl.pallas_call(
        functools.partial(_k3_body, d),
        grid=grid,
        in_specs=[
            pl.BlockSpec((1, kb, d), lambda c, i: (c, i, 0)),
            pl.BlockSpec((1, kb, d), lambda c, i: (c, i, 0)),
            pl.BlockSpec((1, 1, d), lambda c, i: (c, 0, 0)),
            pl.BlockSpec((1, d, d), lambda c, i: (c, 0, 0)),
            pl.BlockSpec((1, 1, d), lambda c, i: (c, 0, 0)),
            pl.BlockSpec((1, d, d), lambda c, i: (c, 0, 0)),
            pl.BlockSpec((1, 1, d), lambda c, i: (c, 0, 0)),
            pl.BlockSpec((1, 1, d), lambda c, i: (c, 0, 0)),
            pl.BlockSpec(memory_space=pltpu.SMEM),
            pl.BlockSpec(memory_space=pltpu.SMEM),
        ],
        out_specs=[
            pl.BlockSpec((1, kb, d), lambda c, i: (c, i, 0)),
            pl.BlockSpec((1, 4, kb), lambda c, i: (c, 0, i)),
        ],
        out_shape=[
            jax.ShapeDtypeStruct((2, e, d), jnp.float32),
            jax.ShapeDtypeStruct((2, 4, e), jnp.float32),
        ],
    )(ga, gb, w1c, w2b, b2s, cw1b, cb1s, cw2t, cb2s, cws)


def _k4m_body(half, rr, e, eh, k, j, mm0, mm1, ia, zm, outm,
              idx_v, idxt_v, m_v, accm):
    c = lax.axis_index("c")
    s = lax.axis_index("s")
    per_sub = eh // NS
    chunks = per_sub // k
    zrows = rr // NS
    wrows = half // NS

    pltpu.sync_copy(zm.at[pl.ds(s * zrows, zrows)],
                    accm.at[pl.ds(s * zrows, zrows)])
    plsc.subcore_barrier()

    def step(t, _):
        base = s * per_sub + t * k
        for ci, mm in ((0, mm0), (1, mm1)):
            pltpu.sync_copy(ia.at[pl.ds(c * e + ci * eh + base, k)], idx_v)
            pltpu.sync_copy(mm.at[c, pl.ds(base, k)], m_v)
            for g in range(k // 16):
                v = idx_v[pl.ds(g * 16, 16)]
                lv = v - (j * half)
                ok = (lv >= 0) & (lv < half)
                idxt_v[pl.ds(g * 16, 16)] = jnp.where(ok, lv, half)
            pltpu.sync_copy(m_v, accm.at[idxt_v], add=True)
        return _

    lax.fori_loop(0, chunks, step, None)
    plsc.subcore_barrier()
    pltpu.sync_copy(accm.at[pl.ds(s * wrows, wrows)],
                    outm.at[c, pl.ds(s * wrows, wrows)])


def _k4x_body(npad, e, eh, k, ww0, ww1, ia, zx4, outx, idx_v, w_v, accx_t):
    c = lax.axis_index("c")
    s = lax.axis_index("s")
    per_sub = eh // NS
    chunks = per_sub // k

    pltpu.sync_copy(zx4, accx_t)

    def step(t, _):
        base = s * per_sub + t * k
        for ci, ww in ((0, ww0), (1, ww1)):
            pltpu.sync_copy(ia.at[pl.ds(c * e + ci * eh + base, k)], idx_v)
            for col in range(3):
                pltpu.sync_copy(ww.at[pl.ds((c * 4 + col) * eh + base, k)],
                                w_v.at[pl.ds(col * k, k)])
            for g in range(k // 16):
                v = idx_v[pl.ds(g * 16, 16)]
                for col in range(3):
                    vals = w_v[pl.ds(col * k + g * 16, 16)]
                    plsc.addupdate_scatter(accx_t, [v * 4 + col], vals)
        return _

    lax.fori_loop(0, chunks, step, None)
    pltpu.sync_copy(accx_t, outx.at[c, s])


def _scatter_stage(mms, wws, ia_flat, n, e, eh, d):
    k = 400
    npad = ((n + NS * 16 - 1) // (NS * 16)) * NS * 16
    half = npad // 2
    rr = half + NS * 8
    zm = jnp.zeros((rr, d), jnp.float32)
    zx4 = jnp.zeros((npad * 4,), jnp.float32)
    mesh = plsc.VectorSubcoreMesh(core_axis_name="c", subcore_axis_name="s")

    halves = []
    for j in (0, 1):
        halves.append(pl.kernel(
            functools.partial(_k4m_body, half, rr, e, eh, k, j),
            out_type=jax.ShapeDtypeStruct((2, half, d), jnp.float32),
            mesh=mesh,
            scratch_types=[
                pltpu.VMEM((k,), jnp.int32),
                pltpu.VMEM((k,), jnp.int32),
                pltpu.VMEM((k, d), jnp.float32),
                pltpu.VMEM_SHARED((rr, d), jnp.float32),
            ],
        )(mms[0], mms[1], ia_flat, zm))

    outx = pl.kernel(
        functools.partial(_k4x_body, npad, e, eh, k),
        out_type=jax.ShapeDtypeStruct((2, NS, npad * 4), jnp.float32),
        mesh=mesh,
        scratch_types=[
            pltpu.VMEM((k,), jnp.int32),
            pltpu.VMEM((4 * k,), jnp.float32),
            pltpu.VMEM((npad * 4,), jnp.float32),
        ],
        compiler_params=pltpu.CompilerParams(needs_layout_passes=False),
    )(wws[0].reshape(2 * 4 * eh), wws[1].reshape(2 * 4 * eh), ia_flat, zx4)

    magg = jnp.concatenate(halves, axis=1)
    return magg, outx.reshape(2 * NS, npad * 4), npad


def _k5h_body(h_ref, ma_ref, w1h_ref, w1u_ref, w1d_ref,
              b1_ref, w2_ref, b2_ref, hout_ref):
    pre = (jnp.dot(h_ref[...], w1h_ref[...], preferred_element_type=jnp.float32)
           + jnp.dot(ma_ref[0], w1u_ref[...], preferred_element_type=jnp.float32)
           + jnp.dot(ma_ref[1], w1d_ref[...], preferred_element_type=jnp.float32)
           + b1_ref[0])
    hout_ref[...] = h_ref[...] + jnp.dot(
        jnp.maximum(pre, 0.0), w2_ref[...],
        preferred_element_type=jnp.float32) + b2_ref[0]


def _node_update_h(h, ma, w1h, w1u, w1d, b1, w2, b2, n, d):
    bn = 2000
    grid = (n // bn,)
    return pl.pallas_call(
        _k5h_body,
        grid=grid,
        in_specs=[
            pl.BlockSpec((bn, d), lambda i: (i, 0)),
            pl.BlockSpec((2, bn, d), lambda i: (0, i, 0)),
            pl.BlockSpec((d, d), lambda i: (0, 0)),
            pl.BlockSpec((d, d), lambda i: (0, 0)),
            pl.BlockSpec((d, d), lambda i: (0, 0)),
            pl.BlockSpec((1, d), lambda i: (0, 0)),
            pl.BlockSpec((d, d), lambda i: (0, 0)),
            pl.BlockSpec((1, d), lambda i: (0, 0)),
        ],
        out_specs=pl.BlockSpec((bn, d), lambda i: (i, 0)),
        out_shape=jax.ShapeDtypeStruct((n, d), jnp.float32),
    )(h, ma, w1h, w1u, w1d, b1, w2, b2)


def _k5x_body(xagg_ref, xpad_ref, xout_ref):
    xout_ref[...] = xpad_ref[...] + jnp.sum(
        xagg_ref[...], axis=0, keepdims=True)


def _node_update_x(xaggf, xpad4f, npad):
    bx = 5120
    grid = (npad * 4 // bx,)
    return pl.pallas_call(
        _k5x_body,
        grid=grid,
        in_specs=[
            pl.BlockSpec((2 * NS, bx), lambda i: (0, i)),
            pl.BlockSpec((1, bx), lambda i: (0, i)),
        ],
        out_specs=pl.BlockSpec((1, bx), lambda i: (0, i)),
        out_shape=jax.ShapeDtypeStruct((1, npad * 4), jnp.float32),
    )(xaggf, xpad4f)


def kernel(h, h_up, h_down, x, x_up, x_down, b_up, b_down, cw,
           bu_w1, bu_b1, bu_w2, bu_b2,
           bd_w1, bd_b1, bd_w2, bd_b2,
           cm_w1, cm_b1, cm_w2, cm_b2,
           cu_w1, cu_b1, cu_w2, cu_b2,
           cd_w1, cd_b1, cd_w2, cd_b2):
    n, d = h.shape
    e = b_up.shape[1]

    hs = jnp.stack([h_up, h_down])
    wa = jnp.stack([bu_w1[:d], bd_w1[:d]])
    wb = jnp.stack([bu_w1[d:2 * d], bd_w1[d:2 * d]])
    w1c = jnp.stack([bu_w1[2 * d], bd_w1[2 * d]]).reshape(2, 1, d)
    b1s = jnp.stack([bu_b1, bd_b1]).reshape(2, 1, d)
    w2b = jnp.stack([bu_w2, bd_w2]).astype(jnp.bfloat16)
    b2s = jnp.stack([bu_b2, bd_b2]).reshape(2, 1, d)
    cw1b = jnp.stack([cu_w1, cd_w1]).astype(jnp.bfloat16)
    cb1s = jnp.stack([cu_b1, cd_b1]).reshape(2, 1, d)
    cw2t = jnp.stack([cu_w2.T, cd_w2.T])
    cb2s = jnp.stack([cu_b2, cd_b2])

    xpad = jnp.pad(x, ((0, 0), (0, XP - 3)))
    xnbs = jnp.stack([jnp.pad(x_up, ((0, 0), (0, XP - 3))),
                      jnp.pad(x_down, ((0, 0), (0, XP - 3)))])

    ia_raw = jnp.stack([b_up[0], b_down[1]])
    ib_raw = jnp.stack([b_up[1], b_down[0]])
    off = jnp.array([[0], [n]], jnp.int32)
    ia_off = (ia_raw + off).reshape(2 * e)
    ib_off = (ib_raw + off).reshape(2 * e)
    ia_flat = ia_raw.reshape(2 * e)

    ta, tb = _make_tables(h, hs, xpad, xnbs, wa, wb, b1s, w1c, n, d)
    tacat = ta.reshape(2 * n, d)
    tbcat = tb.reshape(2 * n, d)

    eh = e // 2
    mms, wws = [], []
    for ci in (0, 1):
        ga, gb = _gather_stage(tacat, tbcat, ia_off, ib_off, e, eh,
                               ci * eh, d)
        mm, ww = _edge_mlp(ga, gb, w1c, w2b, b2s, cw1b, cb1s, cw2t,
                           cb2s, cw, eh, d)
        mms.append(mm)
        wws.append(ww)

    magg, xaggf, npad = _scatter_stage(mms, wws, ia_flat, n, e, eh, d)

    hout = _node_update_h(
        h, magg,
        cm_w1[:d], cm_w1[d:2 * d], cm_w1[2 * d:], cm_b1.reshape(1, d),
        cm_w2, cm_b2.reshape(1, d), n, d)
    xpad4f = jnp.pad(x, ((0, npad - n), (0, 1))).reshape(1, npad * 4)
    xoutf = _node_update_x(xaggf, xpad4f, npad)

    return (hout, xoutf.reshape(npad, 4)[:n, :3])
